# Initial kernel scaffold; baseline (speedup 1.0000x reference)
import jax
import jax.numpy as jnp
from jax.experimental import pallas as pl


def kernel(A, B, k):
    raise NotImplementedError("write your pallas kernel here")


# speedup vs baseline: 8.5846x; 8.5846x over previous
import jax
import jax.numpy as jnp
from jax.experimental import pallas as pl

_M = 4096
_N = 16384
_QT = 256
_NC = 2048


def _nn_body(b_ref, at_ref, dist_ref, idx_ref):
    qx = b_ref[:, 0:1]
    qy = b_ref[:, 1:2]

    def body(j, carry):
        rmin, ridx = carry
        ax = at_ref[0:1, pl.ds(j * _NC, _NC)]
        ay = at_ref[1:2, pl.ds(j * _NC, _NC)]
        dx = qx - ax
        dy = qy - ay
        d2 = dx * dx + dy * dy
        cmin = jnp.min(d2, axis=1, keepdims=True)
        iota = jax.lax.broadcasted_iota(jnp.int32, (_QT, _NC), 1) + j * _NC
        cidx = jnp.min(
            jnp.where(d2 == cmin, iota, jnp.int32(2**30)),
            axis=1, keepdims=True)
        upd = cmin < rmin
        return jnp.where(upd, cmin, rmin), jnp.where(upd, cidx, ridx)

    rmin0 = jnp.full((_QT, 1), jnp.inf, jnp.float32)
    ridx0 = jnp.zeros((_QT, 1), jnp.int32)
    rmin, ridx = jax.lax.fori_loop(0, _N // _NC, body, (rmin0, ridx0))
    dist_ref[:, :] = jnp.sqrt(rmin)
    idx_ref[:, :] = ridx


def kernel(A, B, k):
    AT = A.T
    dist, idx = pl.pallas_call(
        _nn_body,
        grid=(_M // _QT,),
        in_specs=[
            pl.BlockSpec((_QT, 2), lambda i: (i, 0)),
            pl.BlockSpec((2, _N), lambda i: (0, 0)),
        ],
        out_specs=[
            pl.BlockSpec((_QT, 1), lambda i: (i, 0)),
            pl.BlockSpec((_QT, 1), lambda i: (i, 0)),
        ],
        out_shape=[
            jax.ShapeDtypeStruct((_M, 1), jnp.float32),
            jax.ShapeDtypeStruct((_M, 1), jnp.int32),
        ],
    )(B, AT)
    return dist, idx * jnp.asarray(k, dtype=idx.dtype)
